# baseline (speedup 1.0000x reference)
import functools

import jax
import jax.numpy as jnp
from jax import lax
from jax.experimental import pallas as pl
from jax.experimental.pallas import tpu as pltpu
from jax.experimental.pallas import tpu_sc as plsc

NUM_EXPERTS = 64
BR = 2048
LANES = 16


KSPLIT = 2


def _matmul_kernel(w_ref, x_ref, out_ref):
    part = lax.dot_general(
        w_ref[...], x_ref[...],
        dimension_numbers=(((1,), (1,)), ((), ())),
        preferred_element_type=jnp.float32,
    )
    k = pl.program_id(1)

    @pl.when(k == 0)
    def _():
        out_ref[...] = part

    @pl.when(k != 0)
    def _():
        out_ref[...] += part


def _logits_t(x, W, blk_off, rows):
    _, d = x.shape
    dk = d // KSPLIT
    return pl.pallas_call(
        _matmul_kernel,
        grid=(rows // BR, KSPLIT),
        in_specs=[
            pl.BlockSpec((NUM_EXPERTS, dk), lambda i, k: (0, k)),
            pl.BlockSpec((BR, dk), lambda i, k: (i + blk_off, k)),
        ],
        out_specs=pl.BlockSpec((NUM_EXPERTS, BR), lambda i, k: (0, i)),
        out_shape=jax.ShapeDtypeStruct((NUM_EXPERTS, rows), jnp.float32),
    )(W, x)


def _make_topk_sc(n):
    info = plsc.get_sparse_core_info()
    nw = info.num_cores * info.num_subcores
    rows_w = n // nw
    half = rows_w // 2
    groups_h = half // LANES
    mesh = plsc.VectorSubcoreMesh(core_axis_name="c", subcore_axis_name="s")

    @functools.partial(
        pl.kernel, mesh=mesh,
        out_type=[
            jax.ShapeDtypeStruct((n,), jnp.int32),
            jax.ShapeDtypeStruct((n,), jnp.int32),
            jax.ShapeDtypeStruct((n,), jnp.float32),
            jax.ShapeDtypeStruct((n,), jnp.float32),
        ],
        scratch_types=[
            pltpu.VMEM((NUM_EXPERTS, rows_w), jnp.float32),
            pltpu.VMEM((rows_w,), jnp.int32),
            pltpu.VMEM((rows_w,), jnp.int32),
            pltpu.VMEM((rows_w,), jnp.float32),
            pltpu.VMEM((rows_w,), jnp.float32),
            pltpu.SemaphoreType.DMA,
            pltpu.SemaphoreType.DMA,
        ],
    )
    def topk_sc(logits_hbm, i1_hbm, i2_hbm, g1_hbm, g2_hbm,
                slab, i1v, i2v, g1v, g2v, sem0, sem1):
        wid = lax.axis_index("s") * info.num_cores + lax.axis_index("c")
        base = wid * rows_w
        cp0 = pltpu.async_copy(
            logits_hbm.at[:, pl.ds(base, half)], slab.at[:, pl.ds(0, half)], sem0)
        cp1 = pltpu.async_copy(
            logits_hbm.at[:, pl.ds(base + half, half)],
            slab.at[:, pl.ds(half, half)], sem1)

        def scan_one(off):
            m1 = slab[0, pl.ds(off, LANES)]
            i1 = jnp.zeros((LANES,), jnp.int32)
            m2 = jnp.full((LANES,), -jnp.inf, jnp.float32)
            i2 = jnp.zeros((LANES,), jnp.int32)
            for e in range(1, NUM_EXPERTS):
                l = slab[e, pl.ds(off, LANES)]
                gt1 = l > m1
                gt2 = l > m2
                ei = jnp.full((LANES,), e, jnp.int32)
                i2 = jnp.where(gt1, i1, jnp.where(gt2, ei, i2))
                i1 = jnp.where(gt1, ei, i1)
                m2 = jnp.maximum(m2, jnp.minimum(l, m1))
                m1 = jnp.maximum(m1, l)
            ex = jnp.exp(m2 - m1)
            den = 1.0 + ex
            i1v[pl.ds(off, LANES)] = i1
            i2v[pl.ds(off, LANES)] = i2
            g1v[pl.ds(off, LANES)] = 1.0 / den
            g2v[pl.ds(off, LANES)] = ex / den

        def body(g, carry):
            off = g * (2 * LANES)
            scan_one(off)
            scan_one(off + LANES)
            return carry

        pairs_h = groups_h // 2
        cp0.wait()
        lax.fori_loop(0, pairs_h, body, 0)
        cp1.wait()
        lax.fori_loop(pairs_h, 2 * pairs_h, body, 0)
        pltpu.sync_copy(i1v, i1_hbm.at[pl.ds(base, rows_w)])
        pltpu.sync_copy(i2v, i2_hbm.at[pl.ds(base, rows_w)])
        pltpu.sync_copy(g1v, g1_hbm.at[pl.ds(base, rows_w)])
        pltpu.sync_copy(g2v, g2_hbm.at[pl.ds(base, rows_w)])

    return topk_sc


def kernel(x, W):
    n, _ = x.shape
    h = n // 2
    topk = _make_topk_sc(h)
    lt_a = _logits_t(x, W, 0, h)
    lt_b = _logits_t(x, W, h // BR, h)
    ia1, ia2, ga1, ga2 = topk(lt_a)
    ib1, ib2, gb1, gb2 = topk(lt_b)
    i1 = jnp.concatenate([ia1, ib1])
    i2 = jnp.concatenate([ia2, ib2])
    g1 = jnp.concatenate([ga1, gb1])
    g2 = jnp.concatenate([ga2, gb2])
    idx = jnp.concatenate([i1[:, None], i2[:, None]], axis=1)
    gates = jnp.concatenate([g1[:, None], g2[:, None]], axis=1)
    return idx, gates


# speedup vs baseline: 1.0323x; 1.0323x over previous
import functools

import jax
import jax.numpy as jnp
from jax import lax
from jax.experimental import pallas as pl
from jax.experimental.pallas import tpu as pltpu
from jax.experimental.pallas import tpu_sc as plsc

NUM_EXPERTS = 64
BR = 2048
LANES = 16


KSPLIT = 2


def _matmul_kernel(w_ref, x_ref, out_ref):
    part = lax.dot_general(
        w_ref[...], x_ref[...],
        dimension_numbers=(((1,), (1,)), ((), ())),
        preferred_element_type=jnp.float32,
    )
    k = pl.program_id(1)

    @pl.when(k == 0)
    def _():
        out_ref[...] = part

    @pl.when(k != 0)
    def _():
        out_ref[...] += part


def _logits_t(x, W):
    n, d = x.shape
    dk = d // KSPLIT
    return pl.pallas_call(
        _matmul_kernel,
        grid=(n // BR, KSPLIT),
        in_specs=[
            pl.BlockSpec((NUM_EXPERTS, dk), lambda i, k: (0, k)),
            pl.BlockSpec((BR, dk), lambda i, k: (i, k)),
        ],
        out_specs=pl.BlockSpec((NUM_EXPERTS, BR), lambda i, k: (0, i)),
        out_shape=jax.ShapeDtypeStruct((NUM_EXPERTS, n), jnp.float32),
    )(W, x)


def _make_topk_sc(n):
    info = plsc.get_sparse_core_info()
    nw = info.num_cores * info.num_subcores
    rows_w = n // nw
    half = rows_w // 2
    groups_h = half // LANES
    mesh = plsc.VectorSubcoreMesh(core_axis_name="c", subcore_axis_name="s")

    @functools.partial(
        pl.kernel, mesh=mesh,
        out_type=[
            jax.ShapeDtypeStruct((n,), jnp.int32),
            jax.ShapeDtypeStruct((n,), jnp.int32),
            jax.ShapeDtypeStruct((n,), jnp.float32),
            jax.ShapeDtypeStruct((n,), jnp.float32),
        ],
        scratch_types=[
            pltpu.VMEM((NUM_EXPERTS, rows_w), jnp.float32),
            pltpu.VMEM((rows_w,), jnp.int32),
            pltpu.VMEM((rows_w,), jnp.int32),
            pltpu.VMEM((rows_w,), jnp.float32),
            pltpu.VMEM((rows_w,), jnp.float32),
            pltpu.SemaphoreType.DMA,
            pltpu.SemaphoreType.DMA,
        ],
    )
    def topk_sc(logits_hbm, i1_hbm, i2_hbm, g1_hbm, g2_hbm,
                slab, i1v, i2v, g1v, g2v, sem0, sem1):
        wid = lax.axis_index("s") * info.num_cores + lax.axis_index("c")
        base = wid * rows_w
        cp0 = pltpu.async_copy(
            logits_hbm.at[:, pl.ds(base, half)], slab.at[:, pl.ds(0, half)], sem0)
        cp1 = pltpu.async_copy(
            logits_hbm.at[:, pl.ds(base + half, half)],
            slab.at[:, pl.ds(half, half)], sem1)

        def scan_one(off):
            m1 = slab[0, pl.ds(off, LANES)]
            i1 = jnp.zeros((LANES,), jnp.int32)
            m2 = jnp.full((LANES,), -jnp.inf, jnp.float32)
            i2 = jnp.zeros((LANES,), jnp.int32)
            for e in range(1, NUM_EXPERTS):
                l = slab[e, pl.ds(off, LANES)]
                gt1 = l > m1
                gt2 = l > m2
                ei = jnp.full((LANES,), e, jnp.int32)
                i2 = jnp.where(gt1, i1, jnp.where(gt2, ei, i2))
                i1 = jnp.where(gt1, ei, i1)
                m2 = jnp.maximum(m2, jnp.minimum(l, m1))
                m1 = jnp.maximum(m1, l)
            ex = jnp.exp(m2 - m1)
            den = 1.0 + ex
            i1v[pl.ds(off, LANES)] = i1
            i2v[pl.ds(off, LANES)] = i2
            g1v[pl.ds(off, LANES)] = 1.0 / den
            g2v[pl.ds(off, LANES)] = ex / den

        def body(g, carry):
            scan_one(g * LANES)
            return carry

        cp0.wait()
        lax.fori_loop(0, groups_h, body, 0)
        cp1.wait()
        lax.fori_loop(groups_h, 2 * groups_h, body, 0)
        pltpu.sync_copy(i1v, i1_hbm.at[pl.ds(base, rows_w)])
        pltpu.sync_copy(i2v, i2_hbm.at[pl.ds(base, rows_w)])
        pltpu.sync_copy(g1v, g1_hbm.at[pl.ds(base, rows_w)])
        pltpu.sync_copy(g2v, g2_hbm.at[pl.ds(base, rows_w)])

    return topk_sc


def kernel(x, W):
    n, _ = x.shape
    logits_t = _logits_t(x, W)
    i1, i2, g1, g2 = _make_topk_sc(n)(logits_t)
    idx = jnp.concatenate([i1[:, None], i2[:, None]], axis=1)
    gates = jnp.concatenate([g1[:, None], g2[:, None]], axis=1)
    return idx, gates
